# baseline (speedup 1.0000x reference)
import functools

import jax
import jax.numpy as jnp
from jax import lax
from jax.experimental import pallas as pl
from jax.experimental.pallas import tpu as pltpu
from jax.experimental.pallas import tpu_sc as plsc

B = 16384
NC = 2
NS = 16
NW = NC * NS
BPW = B // NW
CHUNK = 16
CW = CHUNK * 8
NCHUNK = BPW // CHUNK
WPW = BPW * 8


def _sc_gather(widxs, tables):
    mesh = plsc.VectorSubcoreMesh(core_axis_name="c", subcore_axis_name="s")

    @functools.partial(
        pl.kernel,
        mesh=mesh,
        out_type=[jax.ShapeDtypeStruct((B * 8,), jnp.float32)] * 9,
        compiler_params=pltpu.CompilerParams(use_tc_tiling_on_sc=False),
        scratch_types=[pltpu.VMEM((9 * WPW,), jnp.int32),
                       pltpu.VMEM((9 * WPW,), jnp.float32),
                       pltpu.SemaphoreType.DMA],
    )
    def k(*refs):
        widx_refs, tabs, outs = refs[0:9], refs[9:18], refs[18:27]
        widx_v, obuf, sem = refs[27], refs[28], refs[29]
        wid = lax.axis_index("s") * NC + lax.axis_index("c")
        base = wid * WPW
        for t in range(9):
            pltpu.sync_copy(widx_refs[t].at[pl.ds(base, WPW)],
                            widx_v.at[pl.ds(t * WPW, WPW)])
        cps = [pltpu.async_copy(
            tabs[t].at[widx_v.at[pl.ds(t * WPW, WPW)]],
            obuf.at[pl.ds(t * WPW, WPW)], sem) for t in range(9)]
        for cp in cps:
            cp.wait()
        for t in range(9):
            pltpu.sync_copy(obuf.at[pl.ds(t * WPW, WPW)],
                            outs[t].at[pl.ds(base, WPW)])

    return k(*widxs, *tables)


def _tc_mlp(numt, embts, w1, b1, w2, b2, w3, b3, wo, bo):
    BLK = 2048

    def body(numt_ref, e0, e1, e2, e3, e4, e5, e6, e7, e8,
             w1_ref, b1_ref, w2_ref, b2_ref, w3_ref, b3_ref, wo_ref, bo_ref,
             out_ref):
        embs = [e[...][:5 if i < 2 else 8, :]
                for i, e in enumerate((e0, e1, e2, e3, e4, e5, e6, e7, e8))]
        x = jnp.concatenate([numt_ref[...]] + embs, axis=0)
        h = jnp.dot(w1_ref[...], x, preferred_element_type=jnp.float32)
        h = jnp.maximum(h + b1_ref[...], 0.0)
        h = jnp.maximum(
            jnp.dot(w2_ref[...], h, preferred_element_type=jnp.float32)
            + b2_ref[...], 0.0)
        h = jnp.maximum(
            jnp.dot(w3_ref[...], h, preferred_element_type=jnp.float32)
            + b3_ref[...], 0.0)
        out_ref[...] = (
            jnp.dot(wo_ref[...], h, preferred_element_type=jnp.float32)
            + bo_ref[...])

    full = lambda shape: pl.BlockSpec(shape, lambda i: (0, 0))
    return pl.pallas_call(
        body,
        grid=(B // BLK,),
        in_specs=[pl.BlockSpec((62, BLK), lambda i: (0, i))]
        + [pl.BlockSpec((8, BLK), lambda i: (0, i)) for _ in range(9)]
        + [full((40, 128)), full((40, 1)),
           full((40, 40)), full((40, 1)),
           full((40, 40)), full((40, 1)),
           full((1, 40)), full((1, 1))],
        out_specs=pl.BlockSpec((1, BLK), lambda i: (0, i)),
        out_shape=jax.ShapeDtypeStruct((1, B), jnp.float32),
    )(numt, *embts, w1, b1, w2, b2, w3, b3, wo, bo)


def kernel(numerical_data, drg_id, aprdrg_id, primary_id, secondary_id,
           third_id, fourth_id, fifth_id, pr1_id, mdc,
           emb_drg, emb_aprdrg, emb_primary, emb_secondary, emb_third,
           emb_fourth, emb_fifth, emb_pr1, emb_mdc,
           fc1_w, fc1_b, fc2_w, fc2_b, fc3_w, fc3_b, out_w, out_b):
    ids = tuple(i.astype(jnp.int32) for i in
                (drg_id, aprdrg_id, primary_id, secondary_id, third_id,
                 fourth_id, fifth_id, pr1_id, mdc))
    pad3 = lambda t: jnp.pad(t, ((0, 0), (0, 3)))
    tables = tuple(t.reshape(-1) for t in
                   (pad3(emb_drg), pad3(emb_aprdrg), emb_primary,
                    emb_secondary, emb_third, emb_fourth, emb_fifth,
                    emb_pr1, emb_mdc))
    d8 = jnp.arange(8, dtype=jnp.int32)
    widxs = tuple(((i * 8)[:, None] + d8[None, :]).reshape(-1) for i in ids)
    flat = _sc_gather(widxs, tables)
    embts = tuple(f.reshape(B, 8).T for f in flat)

    out_t = _tc_mlp(numerical_data.T, embts,
                    fc1_w, fc1_b[:, None], fc2_w, fc2_b[:, None],
                    fc3_w, fc3_b[:, None], out_w, out_b[:, None])
    return out_t.T


# speedup vs baseline: 1.1670x; 1.1670x over previous
import functools

import jax
import jax.numpy as jnp
from jax import lax
from jax.experimental import pallas as pl
from jax.experimental.pallas import tpu as pltpu
from jax.experimental.pallas import tpu_sc as plsc

B = 16384
NC = 2
NS = 16
NW = NC * NS
BPW = B // NW
CHUNK = 256
NCHUNK = BPW // CHUNK
L = 16


def _sc_gather(ids, tables):
    mesh = plsc.VectorSubcoreMesh(core_axis_name="c", subcore_axis_name="s")

    @functools.partial(
        pl.kernel,
        mesh=mesh,
        out_type=[jax.ShapeDtypeStruct((B * L,), jnp.float32)] * 9,
        scratch_types=[pltpu.VMEM((CHUNK,), jnp.int32),
                       pltpu.VMEM((CHUNK,), jnp.int32),
                       pltpu.VMEM((CHUNK, 128), jnp.float32),
                       pltpu.VMEM((CHUNK * L,), jnp.float32),
                       pltpu.SemaphoreType.DMA],
    )
    def k(*refs):
        id_refs, tabs, outs = refs[0:9], refs[9:18], refs[18:27]
        ids_v, rid_v, rows, obuf, sem = refs[27:32]
        wid = lax.axis_index("s") * NC + lax.axis_index("c")
        base = wid * BPW

        for t in range(9):

            def chunk_body(j, carry, id_ref=id_refs[t], tab=tabs[t],
                           out=outs[t]):
                lo = base + j * CHUNK
                pltpu.sync_copy(id_ref.at[pl.ds(lo, CHUNK)], ids_v)

                def rid_body(g, c2):
                    rid_v[pl.ds(g * L, L)] = ids_v[pl.ds(g * L, L)] >> 3
                    return c2

                lax.fori_loop(0, CHUNK // L, rid_body, 0)
                pltpu.async_copy(tab.at[rid_v], rows, sem).wait()

                def ext_body(g, c3):
                    wb16 = (ids_v[pl.ds(g * L, L)] & 7) * L
                    for l in range(L):
                        i = g * L + l
                        obuf[pl.ds(i * L, L)] = rows[i, pl.ds(wb16[l], L)]
                    return c3

                lax.fori_loop(0, CHUNK // L, ext_body, 0)
                pltpu.sync_copy(obuf, out.at[pl.ds(lo * L, CHUNK * L)])
                return carry

            lax.fori_loop(0, NCHUNK, chunk_body, 0)

    return k(*ids, *tables)


def _tc_mlp(numt, embts, w1, b1, w2, b2, w3, b3, wo, bo):
    BLK = 2048

    def body(numt_ref, e0, e1, e2, e3, e4, e5, e6, e7, e8,
             w1_ref, b1_ref, w2_ref, b2_ref, w3_ref, b3_ref, wo_ref, bo_ref,
             out_ref):
        embs = [e[...][:5 if i < 2 else 8, :]
                for i, e in enumerate((e0, e1, e2, e3, e4, e5, e6, e7, e8))]
        x = jnp.concatenate([numt_ref[...]] + embs, axis=0)
        h = jnp.dot(w1_ref[...], x, preferred_element_type=jnp.float32)
        h = jnp.maximum(h + b1_ref[...], 0.0)
        h = jnp.maximum(
            jnp.dot(w2_ref[...], h, preferred_element_type=jnp.float32)
            + b2_ref[...], 0.0)
        h = jnp.maximum(
            jnp.dot(w3_ref[...], h, preferred_element_type=jnp.float32)
            + b3_ref[...], 0.0)
        out_ref[...] = (
            jnp.dot(wo_ref[...], h, preferred_element_type=jnp.float32)
            + bo_ref[...])

    full = lambda shape: pl.BlockSpec(shape, lambda i: (0, 0))
    return pl.pallas_call(
        body,
        grid=(B // BLK,),
        in_specs=[pl.BlockSpec((62, BLK), lambda i: (0, i))]
        + [pl.BlockSpec((8, BLK), lambda i: (0, i)) for _ in range(9)]
        + [full((40, 128)), full((40, 1)),
           full((40, 40)), full((40, 1)),
           full((40, 40)), full((40, 1)),
           full((1, 40)), full((1, 1))],
        out_specs=pl.BlockSpec((1, BLK), lambda i: (0, i)),
        out_shape=jax.ShapeDtypeStruct((1, B), jnp.float32),
    )(numt, *embts, w1, b1, w2, b2, w3, b3, wo, bo)


def _as_rec16(table, vpad):
    v, d = table.shape
    t = jnp.pad(table, ((0, vpad - v), (0, L - d)))
    return t.reshape(vpad // 8, 128)


def kernel(numerical_data, drg_id, aprdrg_id, primary_id, secondary_id,
           third_id, fourth_id, fifth_id, pr1_id, mdc,
           emb_drg, emb_aprdrg, emb_primary, emb_secondary, emb_third,
           emb_fourth, emb_fifth, emb_pr1, emb_mdc,
           fc1_w, fc1_b, fc2_w, fc2_b, fc3_w, fc3_b, out_w, out_b):
    ids = tuple(i.astype(jnp.int32) for i in
                (drg_id, aprdrg_id, primary_id, secondary_id, third_id,
                 fourth_id, fifth_id, pr1_id, mdc))
    tables = (_as_rec16(emb_drg, 1000), _as_rec16(emb_aprdrg, 1000),
              _as_rec16(emb_primary, 100000),
              _as_rec16(emb_secondary, 100000),
              _as_rec16(emb_third, 100000),
              _as_rec16(emb_fourth, 100000),
              _as_rec16(emb_fifth, 100000),
              _as_rec16(emb_pr1, 100000),
              _as_rec16(emb_mdc, 32))
    flat = _sc_gather(ids, tables)
    embts = tuple(f.reshape(B, L)[:, :8].T for f in flat)

    out_t = _tc_mlp(numerical_data.T, embts,
                    fc1_w, fc1_b[:, None], fc2_w, fc2_b[:, None],
                    fc3_w, fc3_b[:, None], out_w, out_b[:, None])
    return out_t.T
